# baseline (speedup 1.0000x reference)
import functools

import jax
import jax.numpy as jnp
from jax import lax
from jax.experimental import pallas as pl
from jax.experimental.pallas import tpu as pltpu
from jax.experimental.pallas import tpu_sc as plsc

_NC = 2
_NS = 16


def _node_pad(n):
    n_pad = -(-n // (_NS * 128)) * (_NS * 128)
    return n_pad + _NS * 128 if n_pad == n else n_pad


def _fill2d(ref, rows, dc, val):
    vec = jnp.full((16,), val, jnp.float32)
    vecs_per_row = dc // 16

    def fbody(i, carry):
        ref[i // vecs_per_row, pl.ds((i % vecs_per_row) * 16, 16)] = vec
        return carry

    lax.fori_loop(0, rows * vecs_per_row, fbody, 0)


@functools.lru_cache(maxsize=None)
def _sc_segment_sum(n, n_chunks, chunk, dc, with_deg):
    assert n_chunks % 4 == 0
    zrows = 32
    n_pad = _node_pad(n)
    rows_per_tile = n_pad // _NS
    n_zcopies = rows_per_tile // zrows
    mesh = plsc.VectorSubcoreMesh(core_axis_name="c", subcore_axis_name="s")

    out_type = [jax.ShapeDtypeStruct((_NC, n_pad, dc), jnp.float32)]
    scratch = [
        pltpu.VMEM((4, chunk), jnp.int32),
        pltpu.VMEM((4, chunk), jnp.int32),
        pltpu.VMEM((chunk, dc), jnp.float32),
        pltpu.VMEM((chunk, dc), jnp.float32),
        pltpu.VMEM((zrows, dc), jnp.float32),
        pltpu.VMEM_SHARED((n_pad, dc), jnp.float32),
        pltpu.SemaphoreType.DMA,
        pltpu.SemaphoreType.DMA,
        pltpu.SemaphoreType.DMA((4,)),
        pltpu.SemaphoreType.DMA((4,)),
    ]
    if with_deg:
        out_type.append(jax.ShapeDtypeStruct((_NC, _NS, n_pad), jnp.float32))
        scratch.append(pltpu.VMEM((n_pad,), jnp.float32))

    @functools.partial(
        pl.kernel, mesh=mesh, out_type=out_type, scratch_types=scratch,
        compiler_params=pltpu.CompilerParams(needs_layout_passes=False))
    def k(table_hbm, src_hbm, dst_hbm, *refs):
        if with_deg:
            (out_hbm, deg_hbm, srcs, dsts, rows_a, rows_b, zero_v, agg_sh,
             gsem_a, gsem_b, isem, dsem, hist) = refs
        else:
            (out_hbm, srcs, dsts, rows_a, rows_b, zero_v, agg_sh,
             gsem_a, gsem_b, isem, dsem) = refs
        cid = lax.axis_index("c")
        sid = lax.axis_index("s")
        wid = sid * _NC + cid

        _fill2d(zero_v, zrows, dc, 0.0)
        row0 = sid * rows_per_tile
        for z in range(n_zcopies):
            pltpu.async_copy(zero_v, agg_sh.at[pl.ds(row0 + z * zrows, zrows)],
                             gsem_a)
        for z in range(n_zcopies):
            pltpu.make_async_copy(
                zero_v, agg_sh.at[pl.ds(row0 + z * zrows, zrows)], gsem_a).wait()
        if with_deg:
            zv = jnp.zeros((16,), jnp.float32)

            def hzero(i, carry):
                hist[pl.ds(i * 16, 16)] = zv
                return carry

            lax.fori_loop(0, n_pad // 16, hzero, 0)
        plsc.subcore_barrier()

        rows = (rows_a, rows_b)
        gsem = (gsem_a, gsem_b)
        ones16 = jnp.ones((16,), jnp.float32)

        pltpu.sync_copy(src_hbm.at[wid, 0], srcs.at[0])
        pltpu.sync_copy(src_hbm.at[wid, 1], srcs.at[1])
        for s in (2, 3):
            pltpu.async_copy(src_hbm.at[wid, s], srcs.at[s], isem.at[s])
        for s in range(4):
            pltpu.async_copy(dst_hbm.at[wid, s], dsts.at[s], dsem.at[s])
        pltpu.async_copy(table_hbm.at[srcs.at[0]], rows_a, gsem_a)
        pltpu.async_copy(table_hbm.at[srcs.at[1]], rows_b, gsem_b)

        def body(j4, carry):
            for u in range(4):
                j = j4 * 4 + u
                b = u % 2
                pltpu.make_async_copy(table_hbm.at[srcs.at[u]], rows[b],
                                      gsem[b]).wait()
                pltpu.make_async_copy(dst_hbm.at[wid, 0], dsts.at[u],
                                      dsem.at[u]).wait()
                pltpu.sync_copy(rows[b], agg_sh.at[dsts.at[u]], add=True)

                @pl.when(j + 2 < n_chunks)
                def _():
                    s2 = (u + 2) % 4
                    pltpu.make_async_copy(src_hbm.at[wid, 0], srcs.at[s2],
                                          isem.at[s2]).wait()
                    pltpu.async_copy(table_hbm.at[srcs.at[s2]], rows[b], gsem[b])

                if with_deg:
                    for g in range(chunk // 16):
                        idx16 = dsts[u, pl.ds(g * 16, 16)]
                        plsc.addupdate_scatter(hist, [idx16], ones16)

                @pl.when(j + 4 < n_chunks)
                def _():
                    pltpu.async_copy(src_hbm.at[wid, j + 4], srcs.at[u],
                                     isem.at[u])
                    pltpu.async_copy(dst_hbm.at[wid, j + 4], dsts.at[u],
                                     dsem.at[u])

            return carry

        lax.fori_loop(0, n_chunks // 4, body, 0)
        plsc.subcore_barrier()
        pltpu.sync_copy(agg_sh.at[pl.ds(row0, rows_per_tile)],
                        out_hbm.at[cid, pl.ds(row0, rows_per_tile)])
        if with_deg:
            pltpu.sync_copy(hist, deg_hbm.at[cid, sid])

    return k


def _logsoftmax(v):
    m = jnp.max(v, axis=-1, keepdims=True)
    s = v - m
    return s - jnp.log(jnp.sum(jnp.exp(s), axis=-1, keepdims=True))


def _tc_layer1_body(x_ref, a_ref, dg_ref, w_ref, b_ref, h_ref, dinv_ref, *, d, bn):
    a = a_ref[0] + a_ref[1]
    deg = jnp.sum(dg_ref[...], axis=(0, 1))[:, None]
    dinv = 1.0 / jnp.maximum(deg, 1.0)
    aggn = a * dinv
    out = (jnp.dot(x_ref[...], w_ref[:d, :], precision=lax.Precision.HIGHEST,
                   preferred_element_type=jnp.float32)
           + jnp.dot(aggn, w_ref[d:, :], precision=lax.Precision.HIGHEST,
                     preferred_element_type=jnp.float32)
           + b_ref[...])
    h_ref[...] = jnp.maximum(out, 0.0)
    dinv_ref[...] = jnp.broadcast_to(dinv, (bn, d))


def _tc_layer_body(h_ref, a_ref, dinv_ref, w_ref, b_ref, o_ref, *, d, last):
    aggn = (a_ref[0] + a_ref[1]) * dinv_ref[...]
    out = (jnp.dot(h_ref[...], w_ref[:d, :], precision=lax.Precision.HIGHEST,
                   preferred_element_type=jnp.float32)
           + jnp.dot(aggn, w_ref[d:, :], precision=lax.Precision.HIGHEST,
                     preferred_element_type=jnp.float32)
           + b_ref[...])
    o_ref[...] = _logsoftmax(out) if last else jnp.maximum(out, 0.0)


def _tc_layer1(x, agg, deg, w, b, *, bn=512):
    n, d = x.shape
    n_pad = deg.shape[2]
    grid = (pl.cdiv(n, bn),)
    return pl.pallas_call(
        functools.partial(_tc_layer1_body, d=d, bn=bn),
        grid=grid,
        in_specs=[
            pl.BlockSpec((bn, d), lambda i: (i, 0)),
            pl.BlockSpec((_NC, bn, d), lambda i: (0, i, 0)),
            pl.BlockSpec((_NC, _NS, bn), lambda i: (0, 0, i)),
            pl.BlockSpec((2 * d, d), lambda i: (0, 0)),
            pl.BlockSpec((1, d), lambda i: (0, 0)),
        ],
        out_specs=[
            pl.BlockSpec((bn, d), lambda i: (i, 0)),
            pl.BlockSpec((bn, d), lambda i: (i, 0)),
        ],
        out_shape=[
            jax.ShapeDtypeStruct((n, d), jnp.float32),
            jax.ShapeDtypeStruct((n, d), jnp.float32),
        ],
    )(x, agg, deg, w, b.reshape(1, d))


def _tc_layer(h, agg, dinv, w, b, *, last, bn=512):
    n, d = h.shape
    grid = (pl.cdiv(n, bn),)
    return pl.pallas_call(
        functools.partial(_tc_layer_body, d=d, last=last),
        grid=grid,
        in_specs=[
            pl.BlockSpec((bn, d), lambda i: (i, 0)),
            pl.BlockSpec((_NC, bn, d), lambda i: (0, i, 0)),
            pl.BlockSpec((bn, d), lambda i: (i, 0)),
            pl.BlockSpec((2 * d, d), lambda i: (0, 0)),
            pl.BlockSpec((1, d), lambda i: (0, 0)),
        ],
        out_specs=pl.BlockSpec((bn, d), lambda i: (i, 0)),
        out_shape=jax.ShapeDtypeStruct((n, d), jnp.float32),
    )(h, agg, dinv, w, b.reshape(1, d))


def kernel(x, edge_index, W1, b1, W2, b2, W3, b3):
    n, d = x.shape
    e = edge_index.shape[1]
    nw = _NC * _NS

    ch1 = 64
    nch1 = -(-e // (nw * ch1) // 4) * 4
    e1 = nw * nch1 * ch1
    src1 = jnp.concatenate(
        [edge_index[0], jnp.zeros((e1 - e,), jnp.int32)]).reshape(nw, nch1, ch1)
    dst1 = jnp.concatenate(
        [edge_index[1], jnp.full((e1 - e,), n, jnp.int32)]).reshape(nw, nch1, ch1)

    ch23 = 100
    nch23 = e // (nw * ch23)
    src23 = edge_index[0].reshape(nw, nch23, ch23)
    dst23 = edge_index[1].reshape(nw, nch23, ch23)

    agg1, deg = _sc_segment_sum(n, nch1, ch1, d, True)(x, src1, dst1)
    h1, dinv = _tc_layer1(x, agg1, deg, W1, b1)
    agg2, = _sc_segment_sum(n, nch23, ch23, d, False)(h1, src23, dst23)
    h2 = _tc_layer(h1, agg2, dinv, W2, b2, last=False)
    agg3, = _sc_segment_sum(n, nch23, ch23, d, False)(h2, src23, dst23)
    return _tc_layer(h2, agg3, dinv, W3, b3, last=True)


# speedup vs baseline: 1.7111x; 1.7111x over previous
import functools

import jax
import jax.numpy as jnp
from jax import lax
from jax.experimental import pallas as pl
from jax.experimental.pallas import tpu as pltpu
from jax.experimental.pallas import tpu_sc as plsc

_NC = 2
_NS = 16
_CHUNK = 100


def _fill_zero(ref, rows, dc, val):
    vec = jnp.full((16,), val, jnp.float32)
    vecs_per_row = dc // 16

    def fbody(i, carry):
        ref[i // vecs_per_row, pl.ds((i % vecs_per_row) * 16, 16)] = vec
        return carry

    lax.fori_loop(0, rows * vecs_per_row, fbody, 0)


_SCHUNK = 50


@functools.lru_cache(maxsize=None)
def _sc_segment_sum(n, e, dc):
    nw = _NC * _NS
    e_per_w = e // nw
    n_chunks = e_per_w // _SCHUNK
    assert n_chunks % 8 == 0 and n_chunks >= 16
    zrows = 32
    n_pad = -(-n // (_NS * 128)) * (_NS * 128)
    rows_per_tile = n_pad // _NS
    n_zcopies = rows_per_tile // zrows
    mesh = plsc.VectorSubcoreMesh(core_axis_name="c", subcore_axis_name="s")

    @functools.partial(
        pl.kernel,
        mesh=mesh,
        out_type=jax.ShapeDtypeStruct((_NC, n_pad, dc), jnp.float32),
        scratch_types=[
            pltpu.VMEM((8, _SCHUNK), jnp.int32),
            pltpu.VMEM((8, _SCHUNK), jnp.int32),
            pltpu.VMEM((_SCHUNK, dc), jnp.float32),
            pltpu.VMEM((_SCHUNK, dc), jnp.float32),
            pltpu.VMEM((_SCHUNK, dc), jnp.float32),
            pltpu.VMEM((_SCHUNK, dc), jnp.float32),
            pltpu.VMEM((zrows, dc), jnp.float32),
            pltpu.VMEM_SHARED((n_pad, dc), jnp.float32),
            pltpu.SemaphoreType.DMA,
            pltpu.SemaphoreType.DMA,
            pltpu.SemaphoreType.DMA,
            pltpu.SemaphoreType.DMA,
            pltpu.SemaphoreType.DMA((8,)),
            pltpu.SemaphoreType.DMA((8,)),
        ],
    )
    def k(table_hbm, src_hbm, dst_hbm, out_hbm,
          srcs, dsts, rows_0, rows_1, rows_2, rows_3, zero_v, agg_sh,
          gsem_0, gsem_1, gsem_2, gsem_3, isem, dsem):
        cid = lax.axis_index("c")
        sid = lax.axis_index("s")
        wid = sid * _NC + cid

        _fill_zero(zero_v, zrows, dc, 0.0)
        row0 = sid * rows_per_tile
        for z in range(n_zcopies):
            pltpu.sync_copy(zero_v, agg_sh.at[pl.ds(row0 + z * zrows, zrows)])
        plsc.subcore_barrier()

        rows = (rows_0, rows_1, rows_2, rows_3)
        gsem = (gsem_0, gsem_1, gsem_2, gsem_3)

        for s in range(8):
            pltpu.async_copy(src_hbm.at[wid, s], srcs.at[s], isem.at[s])
            pltpu.async_copy(dst_hbm.at[wid, s], dsts.at[s], dsem.at[s])
        for t in range(4):
            pltpu.make_async_copy(src_hbm.at[wid, 0], srcs.at[t],
                                  isem.at[t]).wait()
            pltpu.async_copy(table_hbm.at[srcs.at[t]], rows[t], gsem[t])

        def body(j8, carry):
            for u in range(8):
                j = j8 * 8 + u
                b = u % 4
                pltpu.make_async_copy(table_hbm.at[srcs.at[u]], rows[b],
                                      gsem[b]).wait()
                pltpu.make_async_copy(dst_hbm.at[wid, 0], dsts.at[u],
                                      dsem.at[u]).wait()
                pltpu.sync_copy(rows[b], agg_sh.at[dsts.at[u]], add=True)

                @pl.when(j + 4 < n_chunks)
                def _():
                    s4 = (u + 4) % 8
                    pltpu.make_async_copy(src_hbm.at[wid, 0], srcs.at[s4],
                                          isem.at[s4]).wait()
                    pltpu.async_copy(table_hbm.at[srcs.at[s4]], rows[b], gsem[b])

                @pl.when(j + 8 < n_chunks)
                def _():
                    pltpu.async_copy(src_hbm.at[wid, j + 8], srcs.at[u],
                                     isem.at[u])
                    pltpu.async_copy(dst_hbm.at[wid, j + 8], dsts.at[u],
                                     dsem.at[u])

            return carry

        lax.fori_loop(0, n_chunks // 8, body, 0)
        plsc.subcore_barrier()
        pltpu.sync_copy(agg_sh.at[pl.ds(row0, rows_per_tile)],
                        out_hbm.at[cid, pl.ds(row0, rows_per_tile)])

    return k


@functools.lru_cache(maxsize=None)
def _sc_degree(n, e, dc):
    nw = _NC * _NS
    e_per_w = e // nw
    n_chunks = e_per_w // _CHUNK
    zrows = 32
    n_pad = -(-n // (_NS * 128)) * (_NS * 128)
    rows_per_tile = n_pad // _NS
    n_zcopies = rows_per_tile // zrows
    mesh = plsc.VectorSubcoreMesh(core_axis_name="c", subcore_axis_name="s")

    group = 10
    assert n_chunks % group == 0

    @functools.partial(
        pl.kernel,
        mesh=mesh,
        out_type=jax.ShapeDtypeStruct((_NC, n_pad, dc), jnp.float32),
        scratch_types=[
            pltpu.VMEM((n_chunks, _CHUNK), jnp.int32),
            pltpu.VMEM((_CHUNK, dc), jnp.float32),
            pltpu.VMEM((zrows, dc), jnp.float32),
            pltpu.VMEM_SHARED((n_pad, dc), jnp.float32),
            pltpu.SemaphoreType.DMA,
        ],
    )
    def k(dst_hbm, out_hbm, dsts, ones_v, zero_v, deg_sh, sem):
        cid = lax.axis_index("c")
        sid = lax.axis_index("s")
        wid = sid * _NC + cid
        pltpu.sync_copy(dst_hbm.at[wid], dsts)
        _fill_zero(zero_v, zrows, dc, 0.0)
        _fill_zero(ones_v, _CHUNK, dc, 1.0)
        row0 = sid * rows_per_tile
        for z in range(n_zcopies):
            pltpu.sync_copy(zero_v, deg_sh.at[pl.ds(row0 + z * zrows, zrows)])
        plsc.subcore_barrier()

        def body(g, carry):
            j0 = g * group
            for u in range(group):
                pltpu.async_copy(ones_v, deg_sh.at[dsts.at[j0 + u]], sem, add=True)
            for u in range(group):
                pltpu.make_async_copy(ones_v, deg_sh.at[dsts.at[j0 + u]], sem).wait()
            return carry

        lax.fori_loop(0, n_chunks // group, body, 0)
        plsc.subcore_barrier()
        pltpu.sync_copy(deg_sh.at[pl.ds(row0, rows_per_tile)],
                        out_hbm.at[cid, pl.ds(row0, rows_per_tile)])

    return k


def _logsoftmax(v):
    m = jnp.max(v, axis=-1, keepdims=True)
    s = v - m
    return s - jnp.log(jnp.sum(jnp.exp(s), axis=-1, keepdims=True))


def _tc_layer1_body(x_ref, a_ref, dg_ref, w_ref, b_ref, h_ref, dinv_ref, *, d, bn):
    a = a_ref[0] + a_ref[1]
    deg = jnp.max(dg_ref[0] + dg_ref[1], axis=1, keepdims=True)
    dinv = 1.0 / jnp.maximum(deg, 1.0)
    aggn = a * dinv
    out = (jnp.dot(x_ref[...], w_ref[:d, :], precision=lax.Precision.HIGHEST,
                   preferred_element_type=jnp.float32)
           + jnp.dot(aggn, w_ref[d:, :], precision=lax.Precision.HIGHEST,
                     preferred_element_type=jnp.float32)
           + b_ref[...])
    h_ref[...] = jnp.maximum(out, 0.0)
    dinv_ref[...] = jnp.broadcast_to(dinv, (bn, d))


def _tc_layer_body(h_ref, a_ref, dinv_ref, w_ref, b_ref, o_ref, *, d, last):
    aggn = (a_ref[0] + a_ref[1]) * dinv_ref[...]
    out = (jnp.dot(h_ref[...], w_ref[:d, :], precision=lax.Precision.HIGHEST,
                   preferred_element_type=jnp.float32)
           + jnp.dot(aggn, w_ref[d:, :], precision=lax.Precision.HIGHEST,
                     preferred_element_type=jnp.float32)
           + b_ref[...])
    o_ref[...] = _logsoftmax(out) if last else jnp.maximum(out, 0.0)


def _tc_layer1(x, agg, degarr, w, b, *, bn=512):
    n, d = x.shape
    grid = (pl.cdiv(n, bn),)
    return pl.pallas_call(
        functools.partial(_tc_layer1_body, d=d, bn=bn),
        grid=grid,
        in_specs=[
            pl.BlockSpec((bn, d), lambda i: (i, 0)),
            pl.BlockSpec((_NC, bn, d), lambda i: (0, i, 0)),
            pl.BlockSpec((_NC, bn, d), lambda i: (0, i, 0)),
            pl.BlockSpec((2 * d, d), lambda i: (0, 0)),
            pl.BlockSpec((1, d), lambda i: (0, 0)),
        ],
        out_specs=[
            pl.BlockSpec((bn, d), lambda i: (i, 0)),
            pl.BlockSpec((bn, d), lambda i: (i, 0)),
        ],
        out_shape=[
            jax.ShapeDtypeStruct((n, d), jnp.float32),
            jax.ShapeDtypeStruct((n, d), jnp.float32),
        ],
    )(x, agg, degarr, w, b.reshape(1, d))


def _tc_layer(h, agg, dinv, w, b, *, last, bn=512):
    n, d = h.shape
    grid = (pl.cdiv(n, bn),)
    return pl.pallas_call(
        functools.partial(_tc_layer_body, d=d, last=last),
        grid=grid,
        in_specs=[
            pl.BlockSpec((bn, d), lambda i: (i, 0)),
            pl.BlockSpec((_NC, bn, d), lambda i: (0, i, 0)),
            pl.BlockSpec((bn, d), lambda i: (i, 0)),
            pl.BlockSpec((2 * d, d), lambda i: (0, 0)),
            pl.BlockSpec((1, d), lambda i: (0, 0)),
        ],
        out_specs=pl.BlockSpec((bn, d), lambda i: (i, 0)),
        out_shape=jax.ShapeDtypeStruct((n, d), jnp.float32),
    )(h, agg, dinv, w, b.reshape(1, d))


def kernel(x, edge_index, W1, b1, W2, b2, W3, b3):
    n, d = x.shape
    e = edge_index.shape[1]
    nw = _NC * _NS
    ncd = e // (nw * _CHUNK)
    dstd = edge_index[1].reshape(nw, ncd, _CHUNK)
    ncs = e // (nw * _SCHUNK)
    src = edge_index[0].reshape(nw, ncs, _SCHUNK)
    dst = edge_index[1].reshape(nw, ncs, _SCHUNK)

    degarr = _sc_degree(n, e, d)(dstd)
    agg1 = _sc_segment_sum(n, e, d)(x, src, dst)
    h1, dinv = _tc_layer1(x, agg1, degarr, W1, b1)
    agg2 = _sc_segment_sum(n, e, d)(h1, src, dst)
    h2 = _tc_layer(h1, agg2, dinv, W2, b2, last=False)
    agg3 = _sc_segment_sum(n, e, d)(h2, src, dst)
    return _tc_layer(h2, agg3, dinv, W3, b3, last=True)


# speedup vs baseline: 2.0283x; 1.1854x over previous
import functools

import jax
import jax.numpy as jnp
from jax import lax
from jax.experimental import pallas as pl
from jax.experimental.pallas import tpu as pltpu
from jax.experimental.pallas import tpu_sc as plsc

_NC = 2
_NS = 16
_CHUNK = 100


def _fill_zero(ref, rows, dc, val):
    vec = jnp.full((16,), val, jnp.float32)
    vecs_per_row = dc // 16

    def fbody(i, carry):
        ref[i // vecs_per_row, pl.ds((i % vecs_per_row) * 16, 16)] = vec
        return carry

    lax.fori_loop(0, rows * vecs_per_row, fbody, 0)


_SCHUNK = 50


@functools.lru_cache(maxsize=None)
def _sc_segment_sum(n, e, dc):
    nw = _NC * _NS
    e_per_w = e // nw
    n_chunks = e_per_w // _SCHUNK
    assert n_chunks % 8 == 0 and n_chunks >= 16
    zrows = 32
    n_pad = -(-n // (_NS * 128)) * (_NS * 128)
    rows_per_tile = n_pad // _NS
    n_zcopies = rows_per_tile // zrows
    mesh = plsc.VectorSubcoreMesh(core_axis_name="c", subcore_axis_name="s")

    @functools.partial(
        pl.kernel,
        mesh=mesh,
        out_type=jax.ShapeDtypeStruct((_NC, n_pad, dc), jnp.float32),
        scratch_types=[
            pltpu.VMEM((8, _SCHUNK), jnp.int32),
            pltpu.VMEM((8, _SCHUNK), jnp.int32),
            pltpu.VMEM((_SCHUNK, dc), jnp.float32),
            pltpu.VMEM((_SCHUNK, dc), jnp.float32),
            pltpu.VMEM((_SCHUNK, dc), jnp.float32),
            pltpu.VMEM((_SCHUNK, dc), jnp.float32),
            pltpu.VMEM((zrows, dc), jnp.float32),
            pltpu.VMEM_SHARED((n_pad, dc), jnp.float32),
            pltpu.SemaphoreType.DMA,
            pltpu.SemaphoreType.DMA,
            pltpu.SemaphoreType.DMA,
            pltpu.SemaphoreType.DMA,
            pltpu.SemaphoreType.DMA((8,)),
            pltpu.SemaphoreType.DMA((8,)),
        ],
    )
    def k(table_hbm, src_hbm, dst_hbm, out_hbm,
          srcs, dsts, rows_0, rows_1, rows_2, rows_3, zero_v, agg_sh,
          gsem_0, gsem_1, gsem_2, gsem_3, isem, dsem):
        cid = lax.axis_index("c")
        sid = lax.axis_index("s")
        wid = sid * _NC + cid

        _fill_zero(zero_v, zrows, dc, 0.0)
        row0 = sid * rows_per_tile
        for z in range(n_zcopies):
            pltpu.sync_copy(zero_v, agg_sh.at[pl.ds(row0 + z * zrows, zrows)])
        plsc.subcore_barrier()

        rows = (rows_0, rows_1, rows_2, rows_3)
        gsem = (gsem_0, gsem_1, gsem_2, gsem_3)

        for s in range(8):
            pltpu.async_copy(src_hbm.at[wid, s], srcs.at[s], isem.at[s])
            pltpu.async_copy(dst_hbm.at[wid, s], dsts.at[s], dsem.at[s])
        for t in range(4):
            pltpu.make_async_copy(src_hbm.at[wid, 0], srcs.at[t],
                                  isem.at[t]).wait()
            pltpu.async_copy(table_hbm.at[srcs.at[t]], rows[t], gsem[t])

        def body(j8, carry):
            for u in range(8):
                j = j8 * 8 + u
                b = u % 4
                pltpu.make_async_copy(table_hbm.at[srcs.at[u]], rows[b],
                                      gsem[b]).wait()
                pltpu.make_async_copy(dst_hbm.at[wid, 0], dsts.at[u],
                                      dsem.at[u]).wait()
                pltpu.sync_copy(rows[b], agg_sh.at[dsts.at[u]], add=True)

                @pl.when(j + 4 < n_chunks)
                def _():
                    s4 = (u + 4) % 8
                    pltpu.make_async_copy(src_hbm.at[wid, 0], srcs.at[s4],
                                          isem.at[s4]).wait()
                    pltpu.async_copy(table_hbm.at[srcs.at[s4]], rows[b], gsem[b])

                @pl.when(j + 8 < n_chunks)
                def _():
                    pltpu.async_copy(src_hbm.at[wid, j + 8], srcs.at[u],
                                     isem.at[u])
                    pltpu.async_copy(dst_hbm.at[wid, j + 8], dsts.at[u],
                                     dsem.at[u])

            return carry

        lax.fori_loop(0, n_chunks // 8, body, 0)
        plsc.subcore_barrier()
        pltpu.sync_copy(agg_sh.at[pl.ds(row0, rows_per_tile)],
                        out_hbm.at[cid, pl.ds(row0, rows_per_tile)])

    return k


@functools.lru_cache(maxsize=None)
def _sc_degree_hist(n, e):
    nw = _NC * _NS
    e_per_w = e // nw
    assert e_per_w % 16 == 0
    n_pad = _node_pad(n)
    mesh = plsc.VectorSubcoreMesh(core_axis_name="c", subcore_axis_name="s")

    @functools.partial(
        pl.kernel,
        mesh=mesh,
        out_type=jax.ShapeDtypeStruct((_NC, _NS, n_pad), jnp.float32),
        scratch_types=[
            pltpu.VMEM((e_per_w,), jnp.int32),
            pltpu.VMEM((n_pad,), jnp.float32),
        ],
        compiler_params=pltpu.CompilerParams(needs_layout_passes=False),
    )
    def k(dst_hbm, out_hbm, idxs, hist):
        cid = lax.axis_index("c")
        sid = lax.axis_index("s")
        wid = sid * _NC + cid
        pltpu.sync_copy(dst_hbm.at[wid], idxs)
        zv = jnp.zeros((16,), jnp.float32)

        def hzero(i, carry):
            hist[pl.ds(i * 16, 16)] = zv
            return carry

        lax.fori_loop(0, n_pad // 16, hzero, 0)
        ones16 = jnp.ones((16,), jnp.float32)

        def body(i, carry):
            plsc.addupdate_scatter(hist, [idxs[pl.ds(i * 16, 16)]], ones16)
            return carry

        lax.fori_loop(0, e_per_w // 16, body, 0)
        pltpu.sync_copy(hist, out_hbm.at[cid, sid])

    return k


def _node_pad(n):
    n_pad = -(-n // (_NS * 128)) * (_NS * 128)
    return n_pad


def _logsoftmax(v):
    m = jnp.max(v, axis=-1, keepdims=True)
    s = v - m
    return s - jnp.log(jnp.sum(jnp.exp(s), axis=-1, keepdims=True))


def _tc_layer1_body(x_ref, a_ref, dg_ref, w_ref, b_ref, h_ref, dinv_ref, *, d, bn):
    a = a_ref[0] + a_ref[1]
    deg = jnp.sum(dg_ref[0] + dg_ref[1], axis=0)[:, None]
    dinv = 1.0 / jnp.maximum(deg, 1.0)
    aggn = a * dinv
    out = (jnp.dot(x_ref[...], w_ref[:d, :], precision=lax.Precision.HIGHEST,
                   preferred_element_type=jnp.float32)
           + jnp.dot(aggn, w_ref[d:, :], precision=lax.Precision.HIGHEST,
                     preferred_element_type=jnp.float32)
           + b_ref[...])
    h_ref[...] = jnp.maximum(out, 0.0)
    dinv_ref[...] = jnp.broadcast_to(dinv, (bn, d))


def _tc_layer_body(h_ref, a_ref, dinv_ref, w_ref, b_ref, o_ref, *, d, last):
    aggn = (a_ref[0] + a_ref[1]) * dinv_ref[...]
    out = (jnp.dot(h_ref[...], w_ref[:d, :], precision=lax.Precision.HIGHEST,
                   preferred_element_type=jnp.float32)
           + jnp.dot(aggn, w_ref[d:, :], precision=lax.Precision.HIGHEST,
                     preferred_element_type=jnp.float32)
           + b_ref[...])
    o_ref[...] = _logsoftmax(out) if last else jnp.maximum(out, 0.0)


def _tc_layer1(x, agg, degarr, w, b, *, bn=512):
    n, d = x.shape
    grid = (pl.cdiv(n, bn),)
    return pl.pallas_call(
        functools.partial(_tc_layer1_body, d=d, bn=bn),
        grid=grid,
        in_specs=[
            pl.BlockSpec((bn, d), lambda i: (i, 0)),
            pl.BlockSpec((_NC, bn, d), lambda i: (0, i, 0)),
            pl.BlockSpec((_NC, _NS, bn), lambda i: (0, 0, i)),
            pl.BlockSpec((2 * d, d), lambda i: (0, 0)),
            pl.BlockSpec((1, d), lambda i: (0, 0)),
        ],
        out_specs=[
            pl.BlockSpec((bn, d), lambda i: (i, 0)),
            pl.BlockSpec((bn, d), lambda i: (i, 0)),
        ],
        out_shape=[
            jax.ShapeDtypeStruct((n, d), jnp.float32),
            jax.ShapeDtypeStruct((n, d), jnp.float32),
        ],
    )(x, agg, degarr, w, b.reshape(1, d))


def _tc_layer(h, agg, dinv, w, b, *, last, bn=512):
    n, d = h.shape
    grid = (pl.cdiv(n, bn),)
    return pl.pallas_call(
        functools.partial(_tc_layer_body, d=d, last=last),
        grid=grid,
        in_specs=[
            pl.BlockSpec((bn, d), lambda i: (i, 0)),
            pl.BlockSpec((_NC, bn, d), lambda i: (0, i, 0)),
            pl.BlockSpec((bn, d), lambda i: (i, 0)),
            pl.BlockSpec((2 * d, d), lambda i: (0, 0)),
            pl.BlockSpec((1, d), lambda i: (0, 0)),
        ],
        out_specs=pl.BlockSpec((bn, d), lambda i: (i, 0)),
        out_shape=jax.ShapeDtypeStruct((n, d), jnp.float32),
    )(h, agg, dinv, w, b.reshape(1, d))


def kernel(x, edge_index, W1, b1, W2, b2, W3, b3):
    n, d = x.shape
    e = edge_index.shape[1]
    nw = _NC * _NS
    ncs = e // (nw * _SCHUNK)
    src = edge_index[0].reshape(nw, ncs, _SCHUNK)
    dst = edge_index[1].reshape(nw, ncs, _SCHUNK)

    degarr = _sc_degree_hist(n, e)(edge_index[1].reshape(nw, e // nw))
    agg1 = _sc_segment_sum(n, e, d)(x, src, dst)
    h1, dinv = _tc_layer1(x, agg1, degarr, W1, b1)
    agg2 = _sc_segment_sum(n, e, d)(h1, src, dst)
    h2 = _tc_layer(h1, agg2, dinv, W2, b2, last=False)
    agg3 = _sc_segment_sum(n, e, d)(h2, src, dst)
    return _tc_layer(h2, agg3, dinv, W3, b3, last=True)


# speedup vs baseline: 2.0285x; 1.0001x over previous
import functools

import jax
import jax.numpy as jnp
from jax import lax
from jax.experimental import pallas as pl
from jax.experimental.pallas import tpu as pltpu
from jax.experimental.pallas import tpu_sc as plsc

_NC = 2
_NS = 16


def _fill_zero(ref, rows, dc, val):
    vec = jnp.full((16,), val, jnp.float32)
    vecs_per_row = dc // 16

    def fbody(i, carry):
        ref[i // vecs_per_row, pl.ds((i % vecs_per_row) * 16, 16)] = vec
        return carry

    lax.fori_loop(0, rows * vecs_per_row, fbody, 0)


_SCHUNK = 50


@functools.lru_cache(maxsize=None)
def _sc_segment_sum(n, e, dc):
    nw = _NC * _NS
    e_per_w = e // nw
    n_chunks = e_per_w // _SCHUNK
    assert n_chunks % 8 == 0 and n_chunks >= 16
    zrows = 32
    n_pad = -(-n // (_NS * 128)) * (_NS * 128)
    rows_per_tile = n_pad // _NS
    n_zcopies = rows_per_tile // zrows
    mesh = plsc.VectorSubcoreMesh(core_axis_name="c", subcore_axis_name="s")

    @functools.partial(
        pl.kernel,
        mesh=mesh,
        out_type=jax.ShapeDtypeStruct((_NC, n_pad, dc), jnp.float32),
        scratch_types=[
            pltpu.VMEM((8, _SCHUNK), jnp.int32),
            pltpu.VMEM((8, _SCHUNK), jnp.int32),
            pltpu.VMEM((_SCHUNK, dc), jnp.float32),
            pltpu.VMEM((_SCHUNK, dc), jnp.float32),
            pltpu.VMEM((_SCHUNK, dc), jnp.float32),
            pltpu.VMEM((_SCHUNK, dc), jnp.float32),
            pltpu.VMEM((zrows, dc), jnp.float32),
            pltpu.VMEM_SHARED((n_pad, dc), jnp.float32),
            pltpu.SemaphoreType.DMA,
            pltpu.SemaphoreType.DMA,
            pltpu.SemaphoreType.DMA,
            pltpu.SemaphoreType.DMA,
            pltpu.SemaphoreType.DMA((8,)),
            pltpu.SemaphoreType.DMA((8,)),
        ],
    )
    def k(table_hbm, src_hbm, dst_hbm, out_hbm,
          srcs, dsts, rows_0, rows_1, rows_2, rows_3, zero_v, agg_sh,
          gsem_0, gsem_1, gsem_2, gsem_3, isem, dsem):
        cid = lax.axis_index("c")
        sid = lax.axis_index("s")
        wid = sid * _NC + cid

        _fill_zero(zero_v, zrows, dc, 0.0)
        row0 = sid * rows_per_tile
        for z in range(n_zcopies):
            pltpu.sync_copy(zero_v, agg_sh.at[pl.ds(row0 + z * zrows, zrows)])
        plsc.subcore_barrier()

        rows = (rows_0, rows_1, rows_2, rows_3)
        gsem = (gsem_0, gsem_1, gsem_2, gsem_3)

        for s in range(8):
            pltpu.async_copy(src_hbm.at[wid, s], srcs.at[s], isem.at[s])
            pltpu.async_copy(dst_hbm.at[wid, s], dsts.at[s], dsem.at[s])
        for t in range(4):
            pltpu.make_async_copy(src_hbm.at[wid, 0], srcs.at[t],
                                  isem.at[t]).wait()
            pltpu.async_copy(table_hbm.at[srcs.at[t]], rows[t], gsem[t])

        def body(j8, carry):
            for u in range(8):
                j = j8 * 8 + u
                b = u % 4
                pltpu.make_async_copy(table_hbm.at[srcs.at[u]], rows[b],
                                      gsem[b]).wait()
                pltpu.make_async_copy(dst_hbm.at[wid, 0], dsts.at[u],
                                      dsem.at[u]).wait()
                pltpu.sync_copy(rows[b], agg_sh.at[dsts.at[u]], add=True)

                @pl.when(j + 4 < n_chunks)
                def _():
                    s4 = (u + 4) % 8
                    pltpu.make_async_copy(src_hbm.at[wid, 0], srcs.at[s4],
                                          isem.at[s4]).wait()
                    pltpu.async_copy(table_hbm.at[srcs.at[s4]], rows[b], gsem[b])

                @pl.when(j + 8 < n_chunks)
                def _():
                    pltpu.async_copy(src_hbm.at[wid, j + 8], srcs.at[u],
                                     isem.at[u])
                    pltpu.async_copy(dst_hbm.at[wid, j + 8], dsts.at[u],
                                     dsem.at[u])

            return carry

        lax.fori_loop(0, n_chunks // 8, body, 0)
        plsc.subcore_barrier()
        pltpu.sync_copy(agg_sh.at[pl.ds(row0, rows_per_tile)],
                        out_hbm.at[cid, pl.ds(row0, rows_per_tile)])

    return k


@functools.lru_cache(maxsize=None)
def _sc_degree_hist(n, e):
    nw = _NC * _NS
    e_per_w = e // nw
    assert e_per_w % 16 == 0
    n_pad = _node_pad(n)
    mesh = plsc.VectorSubcoreMesh(core_axis_name="c", subcore_axis_name="s")

    @functools.partial(
        pl.kernel,
        mesh=mesh,
        out_type=jax.ShapeDtypeStruct((_NC, _NS, n_pad), jnp.float32),
        scratch_types=[
            pltpu.VMEM((e_per_w,), jnp.int32),
            pltpu.VMEM((n_pad,), jnp.float32),
        ],
        compiler_params=pltpu.CompilerParams(needs_layout_passes=False),
    )
    def k(dst_hbm, out_hbm, idxs, hist):
        cid = lax.axis_index("c")
        sid = lax.axis_index("s")
        wid = sid * _NC + cid
        pltpu.sync_copy(dst_hbm.at[wid], idxs)
        zv = jnp.zeros((16,), jnp.float32)

        def hzero(i, carry):
            hist[pl.ds(i * 16, 16)] = zv
            return carry

        lax.fori_loop(0, n_pad // 16, hzero, 0)
        ones16 = jnp.ones((16,), jnp.float32)

        def body(i, carry):
            plsc.addupdate_scatter(hist, [idxs[pl.ds(i * 16, 16)]], ones16)
            return carry

        lax.fori_loop(0, e_per_w // 16, body, 0)
        pltpu.sync_copy(hist, out_hbm.at[cid, sid])

    return k


def _node_pad(n):
    return -(-n // (_NS * 128)) * (_NS * 128)


def _logsoftmax(v):
    m = jnp.max(v, axis=-1, keepdims=True)
    s = v - m
    return s - jnp.log(jnp.sum(jnp.exp(s), axis=-1, keepdims=True))


def _tc_layer1_body(x_ref, a_ref, dg_ref, w_ref, b_ref, h_ref, dinv_ref, *, d, bn):
    a = a_ref[0] + a_ref[1]
    deg = jnp.sum(dg_ref[0] + dg_ref[1], axis=0)[:, None]
    dinv = 1.0 / jnp.maximum(deg, 1.0)
    aggn = a * dinv
    out = (jnp.dot(x_ref[...], w_ref[:d, :], precision=lax.Precision.HIGHEST,
                   preferred_element_type=jnp.float32)
           + jnp.dot(aggn, w_ref[d:, :], precision=lax.Precision.HIGHEST,
                     preferred_element_type=jnp.float32)
           + b_ref[...])
    h_ref[...] = jnp.maximum(out, 0.0)
    dinv_ref[...] = jnp.broadcast_to(dinv, (bn, d))


def _tc_layer_body(h_ref, a_ref, dinv_ref, w_ref, b_ref, o_ref, *, d, last):
    aggn = (a_ref[0] + a_ref[1]) * dinv_ref[...]
    out = (jnp.dot(h_ref[...], w_ref[:d, :], precision=lax.Precision.HIGHEST,
                   preferred_element_type=jnp.float32)
           + jnp.dot(aggn, w_ref[d:, :], precision=lax.Precision.HIGHEST,
                     preferred_element_type=jnp.float32)
           + b_ref[...])
    o_ref[...] = _logsoftmax(out) if last else jnp.maximum(out, 0.0)


def _tc_layer1(x, agg, degarr, w, b, *, bn=512):
    n, d = x.shape
    grid = (pl.cdiv(n, bn),)
    return pl.pallas_call(
        functools.partial(_tc_layer1_body, d=d, bn=bn),
        grid=grid,
        in_specs=[
            pl.BlockSpec((bn, d), lambda i: (i, 0)),
            pl.BlockSpec((_NC, bn, d), lambda i: (0, i, 0)),
            pl.BlockSpec((_NC, _NS, bn), lambda i: (0, 0, i)),
            pl.BlockSpec((2 * d, d), lambda i: (0, 0)),
            pl.BlockSpec((1, d), lambda i: (0, 0)),
        ],
        out_specs=[
            pl.BlockSpec((bn, d), lambda i: (i, 0)),
            pl.BlockSpec((bn, d), lambda i: (i, 0)),
        ],
        out_shape=[
            jax.ShapeDtypeStruct((n, d), jnp.float32),
            jax.ShapeDtypeStruct((n, d), jnp.float32),
        ],
    )(x, agg, degarr, w, b.reshape(1, d))


def _tc_layer(h, agg, dinv, w, b, *, last, bn=512):
    n, d = h.shape
    grid = (pl.cdiv(n, bn),)
    return pl.pallas_call(
        functools.partial(_tc_layer_body, d=d, last=last),
        grid=grid,
        in_specs=[
            pl.BlockSpec((bn, d), lambda i: (i, 0)),
            pl.BlockSpec((_NC, bn, d), lambda i: (0, i, 0)),
            pl.BlockSpec((bn, d), lambda i: (i, 0)),
            pl.BlockSpec((2 * d, d), lambda i: (0, 0)),
            pl.BlockSpec((1, d), lambda i: (0, 0)),
        ],
        out_specs=pl.BlockSpec((bn, d), lambda i: (i, 0)),
        out_shape=jax.ShapeDtypeStruct((n, d), jnp.float32),
    )(h, agg, dinv, w, b.reshape(1, d))


def kernel(x, edge_index, W1, b1, W2, b2, W3, b3):
    n, d = x.shape
    e = edge_index.shape[1]
    nw = _NC * _NS
    ncs = e // (nw * _SCHUNK)
    src = edge_index[0].reshape(nw, ncs, _SCHUNK)
    dst = edge_index[1].reshape(nw, ncs, _SCHUNK)

    degarr = _sc_degree_hist(n, e)(edge_index[1].reshape(nw, e // nw))
    agg1 = _sc_segment_sum(n, e, d)(x, src, dst)
    h1, dinv = _tc_layer1(x, agg1, degarr, W1, b1)
    agg2 = _sc_segment_sum(n, e, d)(h1, src, dst)
    h2 = _tc_layer(h1, agg2, dinv, W2, b2, last=False)
    agg3 = _sc_segment_sum(n, e, d)(h2, src, dst)
    return _tc_layer(h2, agg3, dinv, W3, b3, last=True)
